# Initial kernel scaffold; baseline (speedup 1.0000x reference)
import jax
import jax.numpy as jnp
from jax.experimental import pallas as pl


def kernel(x, edge_index, edge_weight, graph_ids, Cl, ge, idx, mlp_W1, mlp_b1, mlp_g1, mlp_be1, mlp_W2, mlp_b2, bn_g, bn_b, pred_W, pred_b):
    raise NotImplementedError("write your pallas kernel here")


# speedup vs baseline: 2.8724x; 2.8724x over previous
import functools

import jax
import jax.numpy as jnp
from jax import lax
from jax.experimental import pallas as pl
from jax.experimental.pallas import tpu as pltpu
from jax.experimental.pallas import tpu_sc as plsc

NC = 2
NS = 16
NW = NC * NS
LANES = 16
K = 128

F32 = jnp.float32
I32 = jnp.int32


@functools.cache
def _build_spmm(n_pad, d, n_chunks, interpret=False):
    ntc = NW * n_chunks
    rpt = n_pad // NS
    mesh = plsc.VectorSubcoreMesh(core_axis_name="c", subcore_axis_name="s",
                                  num_cores=NC, num_subcores=NS)

    def body(h, srcm, dstm, wrep, zeros, out, src_v, dst_v, w_v, rows_v,
             pooled_sh, sem):
        c = lax.axis_index("c")
        s = lax.axis_index("s")
        wid = s * NC + c

        pltpu.sync_copy(zeros.at[pl.ds(s * rpt, rpt)],
                        pooled_sh.at[pl.ds(s * rpt, rpt)])
        plsc.subcore_barrier()

        base = wid * n_chunks

        def chunk_body(t, carry):
            row = base + t
            pltpu.sync_copy(srcm.at[row], src_v.at[0])
            pltpu.sync_copy(dstm.at[row], dst_v.at[0])
            pltpu.sync_copy(wrep.at[row], w_v.at[0])
            pltpu.async_copy(h.at[src_v.at[0]], rows_v.at[0], sem).wait()

            def edge_body(j, carry2):
                wv = w_v[0, j]
                for cb in range(d // LANES):
                    sl = pl.ds(cb * LANES, LANES)
                    rows_v[0, j, sl] = rows_v[0, j, sl] * wv
                return carry2

            lax.fori_loop(0, K, edge_body, 0)
            pltpu.sync_copy(rows_v.at[0], pooled_sh.at[dst_v.at[0]], add=True)
            return carry

        lax.fori_loop(0, n_chunks, chunk_body, 0)
        plsc.subcore_barrier()
        pltpu.sync_copy(pooled_sh.at[pl.ds(s * rpt, rpt)],
                        out.at[c, pl.ds(s * rpt, rpt)])

    return pl.kernel(
        body,
        out_type=jax.ShapeDtypeStruct((NC, n_pad, d), F32),
        mesh=mesh,
        scratch_types=[
            pltpu.VMEM((1, K), I32),
            pltpu.VMEM((1, K), I32),
            pltpu.VMEM((1, K, LANES), F32),
            pltpu.VMEM((1, K, d), F32),
            pltpu.VMEM_SHARED((n_pad, d), F32),
            pltpu.SemaphoreType.DMA,
        ],
        interpret=interpret,
    )


@functools.cache
def _build_prep(n_pad, d, g, m, cdim, bn, interpret=False):
    nb = n_pad // bn

    def body(gids_ref, x_ref, cl_ref, ge_ref, idx_ref,
             P_ref, bias_ref, ph0_ref, tmp_ref, ci_s, tmp_s):
        i = pl.program_id(0)

        @pl.when(i == 0)
        def _():
            def gather_row(k, carry):
                ci_s[pl.ds(k, 1), :] = cl_ref[pl.ds(idx_ref[k], 1), :]
                return carry
            lax.fori_loop(0, g, gather_row, 0)
            t1 = lax.dot_general(ci_s[...], cl_ref[...],
                                 (((1,), (1,)), ((), ())),
                                 preferred_element_type=F32)
            t2 = lax.dot_general(t1, ge_ref[...],
                                 (((1,), (0,)), ((), ())),
                                 preferred_element_type=F32)
            tmp_s[...] = t2
            tmp_ref[...] = t2
            ph0_ref[...] = jnp.zeros((g, d), F32)

        ids = gids_ref[0, 0, :]
        iota_g = lax.broadcasted_iota(I32, (g, bn), 0)
        P_blk = (iota_g == ids[None, :]).astype(F32)
        P_ref[...] = P_blk
        bias_ref[...] = lax.dot_general(P_blk, tmp_s[...],
                                        (((0,), (0,)), ((), ())),
                                        preferred_element_type=F32)
        ph0_ref[...] += lax.dot_general(P_blk, x_ref[...],
                                        (((1,), (0,)), ((), ())),
                                        preferred_element_type=F32)

    return pl.pallas_call(
        body,
        grid=(nb,),
        in_specs=[
            pl.BlockSpec((1, 1, bn), lambda i: (i, 0, 0)),
            pl.BlockSpec((bn, d), lambda i: (i, 0)),
            pl.BlockSpec((m, cdim), lambda i: (0, 0)),
            pl.BlockSpec((m, d), lambda i: (0, 0)),
            pl.BlockSpec(memory_space=pltpu.SMEM),
        ],
        out_specs=[
            pl.BlockSpec((g, bn), lambda i: (0, i)),
            pl.BlockSpec((bn, d), lambda i: (i, 0)),
            pl.BlockSpec((g, d), lambda i: (0, 0)),
            pl.BlockSpec((g, d), lambda i: (0, 0)),
        ],
        out_shape=[
            jax.ShapeDtypeStruct((g, n_pad), F32),
            jax.ShapeDtypeStruct((n_pad, d), F32),
            jax.ShapeDtypeStruct((g, d), F32),
            jax.ShapeDtypeStruct((g, d), F32),
        ],
        scratch_shapes=[
            pltpu.VMEM((g, cdim), F32),
            pltpu.VMEM((g, d), F32),
        ],
        interpret=interpret,
    )


@functools.cache
def _build_layer(n_pad, d, g, bn, interpret=False):
    nb = n_pad // bn

    def body(parts_ref0, parts_ref1, bias_ref, P_ref,
             w1_ref, b1_ref, g1_ref, be1_ref, w2_ref, b2_ref, bng_ref, bnb_ref,
             h_ref, ph_ref):
        i = pl.program_id(0)
        pooled = parts_ref0[0] + parts_ref1[0] + bias_ref[...]
        h1 = lax.dot_general(pooled, w1_ref[...], (((1,), (1,)), ((), ())),
                             preferred_element_type=F32) + b1_ref[...]
        h1 = jnp.maximum(h1 * g1_ref[...] + be1_ref[...], 0.0)
        pr = lax.dot_general(h1, w2_ref[...], (((1,), (1,)), ((), ())),
                             preferred_element_type=F32) + b2_ref[...]
        h = jnp.maximum(pr * bng_ref[...] + bnb_ref[...], 0.0)
        h_ref[...] = h

        @pl.when(i == 0)
        def _():
            ph_ref[...] = jnp.zeros((g, d), F32)
        ph_ref[...] += lax.dot_general(P_ref[...], h, (((1,), (0,)), ((), ())),
                                       preferred_element_type=F32)

    vec = pl.BlockSpec((1, d), lambda i: (0, 0))
    return pl.pallas_call(
        body,
        grid=(nb,),
        in_specs=[
            pl.BlockSpec((1, bn, d), lambda i: (0, i, 0)),
            pl.BlockSpec((1, bn, d), lambda i: (1, i, 0)),
            pl.BlockSpec((bn, d), lambda i: (i, 0)),
            pl.BlockSpec((g, bn), lambda i: (0, i)),
            pl.BlockSpec((d, d), lambda i: (0, 0)),
            vec, vec, vec,
            pl.BlockSpec((d, d), lambda i: (0, 0)),
            vec, vec, vec,
        ],
        out_specs=[
            pl.BlockSpec((bn, d), lambda i: (i, 0)),
            pl.BlockSpec((g, d), lambda i: (0, 0)),
        ],
        out_shape=[
            jax.ShapeDtypeStruct((n_pad, d), F32),
            jax.ShapeDtypeStruct((g, d), F32),
        ],
        interpret=interpret,
    )


@functools.cache
def _build_readout(g, d, out_dim, n_layers, interpret=False):
    def body(ph_ref, pw_ref, pb_ref, score_ref):
        acc = jnp.zeros((g, out_dim), F32)
        for l in range(n_layers):
            acc += lax.dot_general(ph_ref[l], pw_ref[l],
                                   (((1,), (1,)), ((), ())),
                                   preferred_element_type=F32)
        acc += jnp.sum(pb_ref[...], axis=0)[None, :]
        score_ref[...] = acc

    return pl.pallas_call(
        body,
        out_shape=jax.ShapeDtypeStruct((g, out_dim), F32),
        interpret=interpret,
    )


def _forward_impl(x, edge_index, edge_weight, graph_ids, Cl, ge, idx,
                  mlp_W1, mlp_b1, mlp_g1, mlp_be1, mlp_W2, mlp_b2,
                  bn_g, bn_b, pred_W, pred_b, interpret=False):
    n, d = x.shape
    e = edge_weight.shape[0]
    g = idx.shape[0]
    m, cdim = Cl.shape
    n_layers = mlp_W1.shape[0]
    out_dim = pred_W.shape[1]

    bn = 1024
    n_pad = ((n + bn - 1) // bn) * bn
    e_per_w = -(-e // NW)
    n_chunks = -(-e_per_w // K)
    e_pad = NW * n_chunks * K

    dst = edge_index[0]
    src = edge_index[1]
    pad_e = e_pad - e
    src_p = jnp.concatenate([src, jnp.zeros((pad_e,), I32)])
    dst_p = jnp.concatenate([dst, (jnp.arange(pad_e, dtype=I32) % n)])
    w_p = jnp.concatenate([edge_weight, jnp.zeros((pad_e,), F32)])

    ntc = e_pad // K
    src2 = src_p.reshape(ntc, K)
    dst2 = dst_p.reshape(ntc, K)
    wrep = jnp.broadcast_to(w_p[:, None], (e_pad, LANES)).reshape(ntc, K, LANES)

    x_pad = jnp.concatenate([x, jnp.zeros((n_pad - n, d), F32)])
    gids_pad = jnp.concatenate(
        [graph_ids, jnp.full((n_pad - n,), g, I32)]).reshape(n_pad // bn, 1, bn)
    zeros_nd = jnp.zeros((n_pad, d), F32)

    spmm = _build_spmm(n_pad, d, n_chunks, interpret)
    prep = _build_prep(n_pad, d, g, m, cdim, bn, interpret)
    layer = _build_layer(n_pad, d, g, bn, interpret)
    readout = _build_readout(g, d, out_dim, n_layers + 1, interpret)

    P, bias, ph0, _tmp = prep(gids_pad, x_pad, Cl, ge, idx)
    phs = [ph0]
    h = x_pad
    for l in range(n_layers):
        parts = spmm(h, src2, dst2, wrep, zeros_nd)
        h, ph_l = layer(parts, parts, bias, P,
                        mlp_W1[l], mlp_b1[l][None, :], mlp_g1[l][None, :],
                        mlp_be1[l][None, :], mlp_W2[l], mlp_b2[l][None, :],
                        bn_g[l][None, :], bn_b[l][None, :])
        phs.append(ph_l)
    score = readout(jnp.stack(phs), pred_W, pred_b)
    return score


def kernel(x, edge_index, edge_weight, graph_ids, Cl, ge, idx,
           mlp_W1, mlp_b1, mlp_g1, mlp_be1, mlp_W2, mlp_b2,
           bn_g, bn_b, pred_W, pred_b):
    return _forward_impl(x, edge_index, edge_weight, graph_ids, Cl, ge, idx,
                         mlp_W1, mlp_b1, mlp_g1, mlp_be1, mlp_W2, mlp_b2,
                         bn_g, bn_b, pred_W, pred_b)
